# baseline (speedup 1.0000x reference)
import functools

import jax
import jax.numpy as jnp
from jax import lax
from jax.experimental import pallas as pl
from jax.experimental.pallas import tpu as pltpu
from jax.experimental.pallas import tpu_sc as plsc

NC = 2
NS = 16
L = 16
B = 128


@functools.lru_cache(maxsize=None)
def _make_seg_kernel(N, Np, Dh, EP):
    RPT = Np // NS
    RA = RPT + 8
    CH = 2048
    NCH = EP // CH
    assert EP % CH == 0
    LCAP = CH + B + 2 * L
    mesh = plsc.VectorSubcoreMesh(
        core_axis_name="c", subcore_axis_name="s",
        num_cores=NC, num_subcores=NS)

    @functools.partial(
        pl.kernel,
        out_type=[
            jax.ShapeDtypeStruct((NC, Np, Dh), jnp.float32),
            jax.ShapeDtypeStruct((NC, Np), jnp.float32),
        ],
        mesh=mesh,
        scratch_types=[
            pltpu.VMEM((2, CH), jnp.int32),
            pltpu.VMEM((2, CH), jnp.int32),
            pltpu.VMEM((LCAP,), jnp.int32),
            pltpu.VMEM((LCAP,), jnp.int32),
            pltpu.VMEM((2, B, Dh), jnp.float32),
            pltpu.VMEM((RA, Dh), jnp.float32),
            pltpu.VMEM((RA,), jnp.float32),
            pltpu.SemaphoreType.DMA,
            pltpu.SemaphoreType.DMA,
        ],
        compiler_params=pltpu.CompilerParams(
            needs_layout_passes=False, use_tc_tiling_on_sc=False),
    )
    def seg(h2_hbm, srcf_hbm, dstf_hbm, agg_hbm, deg_hbm,
            dbuf, sbuf, ldl, sgl, stage, acc, degv, sem, sem2):
        c = lax.axis_index("c")
        s = lax.axis_index("s")
        lo = s * RPT
        z16 = jnp.zeros((L,), jnp.float32)
        ones16 = jnp.ones((L,), jnp.float32)
        pad16 = jnp.full((L,), RPT + 1, jnp.int32)
        zero16i = jnp.zeros((L,), jnp.int32)

        def zero_acc(i, carry):
            for j in range(Dh // L):
                acc[i, pl.ds(j * L, L)] = z16
            return carry
        lax.fori_loop(0, RA, zero_acc, 0)

        def zero_deg(i, carry):
            degv[pl.ds(i * L, L)] = z16
            return carry
        lax.fori_loop(0, RA // L, zero_deg, 0)

        col = lax.iota(jnp.int32, L)
        cols = [col + g * L for g in range(Dh // L)]

        def issue_gather(b, buf):
            return pltpu.async_copy(
                h2_hbm.at[c].at[sgl.at[pl.ds(b * B, B)]], stage.at[buf],
                sem)

        def drain(cnt, nb):
            @pl.when(nb > 0)
            def _():
                issue_gather(0, 0)

            def batch(b, carry):
                buf = lax.rem(b, 2)
                for l in range(B // L):
                    dvec = ldl[pl.ds(b * B + l * L, L)]
                    plsc.addupdate_scatter(degv, [dvec], ones16)
                pltpu.make_async_copy(
                    h2_hbm.at[c].at[sgl.at[pl.ds(0, B)]], stage.at[buf],
                    sem).wait()

                @pl.when(b + 1 < nb)
                def _():
                    issue_gather(b + 1, 1 - buf)

                ebase = jnp.full((L,), b * B, jnp.int32)

                def edge(i, carry2):
                    for u in range(4):
                        e = i * 4 + u
                        rvec = plsc.load_gather(ldl, [ebase + e])
                        for g in range(Dh // L):
                            vals = stage[buf, e, pl.ds(g * L, L)]
                            plsc.addupdate_scatter(acc, [rvec, cols[g]], vals)
                    return carry2
                lax.fori_loop(0, B // 4, edge, 0)
                return carry
            lax.fori_loop(0, nb, batch, 0)
            for t in range(B // L):
                ldl[pl.ds(t * L, L)] = ldl[pl.ds(nb * B + t * L, L)]
                sgl[pl.ds(t * L, L)] = sgl[pl.ds(nb * B + t * L, L)]
            return cnt - nb * B

        def issue_chunk(ch, buf):
            pltpu.async_copy(dstf_hbm.at[pl.ds(ch * CH, CH)], dbuf.at[buf],
                             sem2)
            pltpu.async_copy(srcf_hbm.at[pl.ds(ch * CH, CH)], sbuf.at[buf],
                             sem2)

        issue_chunk(0, 0)

        def chunk(ch, cnt):
            buf = lax.rem(ch, 2)
            pltpu.make_async_copy(dstf_hbm.at[pl.ds(0, CH)], dbuf.at[buf],
                                  sem2).wait()
            pltpu.make_async_copy(srcf_hbm.at[pl.ds(0, CH)], sbuf.at[buf],
                                  sem2).wait()

            @pl.when(ch + 1 < NCH)
            def _():
                issue_chunk(ch + 1, 1 - buf)

            def scan(v, cnt):
                for u in range(2):
                    off = (v * 2 + u) * L
                    dvec = dbuf[buf, pl.ds(off, L)]
                    svec = sbuf[buf, pl.ds(off, L)]
                    ld = dvec - lo
                    m = (ld >= 0) & (ld < RPT)
                    plsc.store_compressed(ldl.at[pl.ds(cnt, L)], ld, mask=m)
                    plsc.store_compressed(sgl.at[pl.ds(cnt, L)], svec, mask=m)
                    cnt = cnt + jnp.max(plsc.all_reduce_population_count(m))
                return cnt
            cnt = lax.fori_loop(0, CH // L // 2, scan, cnt)
            return drain(cnt, cnt // B)
        cnt = lax.fori_loop(0, NCH, chunk, 0)

        for t in range(B // L):
            ldl[pl.ds(cnt + t * L, L)] = pad16
            sgl[pl.ds(cnt + t * L, L)] = zero16i
        drain(cnt, (cnt + B - 1) // B)

        pltpu.sync_copy(acc.at[pl.ds(0, RPT)],
                        agg_hbm.at[c, pl.ds(lo, RPT)])
        pltpu.sync_copy(degv.at[pl.ds(0, RPT)],
                        deg_hbm.at[c, pl.ds(lo, RPT)])

    return seg


@functools.lru_cache(maxsize=None)
def _make_score_kernel(N, E):
    NW = NC * NS
    EPT = E // NW
    assert EPT % 8 == 0 and EPT % L == 0
    mesh = plsc.VectorSubcoreMesh(
        core_axis_name="c", subcore_axis_name="s",
        num_cores=NC, num_subcores=NS)

    @functools.partial(
        pl.kernel,
        out_type=jax.ShapeDtypeStruct((E,), jnp.float32),
        mesh=mesh,
        scratch_types=[
            pltpu.VMEM((N, 2), jnp.float32),
            pltpu.VMEM((EPT,), jnp.int32),
            pltpu.VMEM((EPT,), jnp.int32),
            pltpu.VMEM((EPT,), jnp.float32),
        ],
        compiler_params=pltpu.CompilerParams(
            needs_layout_passes=False, use_tc_tiling_on_sc=False),
    )
    def score(uv_hbm, src_hbm, dst_hbm, out_hbm, uv, sv, dv, ov):
        c = lax.axis_index("c")
        s = lax.axis_index("s")
        wid = s * NC + c
        pltpu.sync_copy(uv_hbm, uv)
        pltpu.sync_copy(src_hbm.at[pl.ds(wid * EPT, EPT)], sv)
        pltpu.sync_copy(dst_hbm.at[pl.ds(wid * EPT, EPT)], dv)
        zc = jnp.zeros((L,), jnp.int32)
        oc = jnp.ones((L,), jnp.int32)

        def body(i, carry):
            si = sv[pl.ds(i * L, L)]
            di = dv[pl.ds(i * L, L)]
            us = plsc.load_gather(uv, [si, zc])
            vd = plsc.load_gather(uv, [di, oc])
            ov[pl.ds(i * L, L)] = us + vd
            return carry
        lax.fori_loop(0, EPT // L, body, 0)
        pltpu.sync_copy(ov, out_hbm.at[pl.ds(wid * EPT, EPT)])

    return score


def _deg_reduce(degp):
    NCc, Np = degp.shape

    def body(degp_ref, invd_ref):
        deg = (degp_ref[0] + degp_ref[1]) * 0.5
        invd_ref[...] = (1.0 / jnp.maximum(deg, 1.0))[:, None]

    return pl.pallas_call(
        body,
        in_specs=[pl.BlockSpec((NCc, Np), lambda: (0, 0))],
        out_specs=pl.BlockSpec((Np, 1), lambda: (0, 0)),
        out_shape=jax.ShapeDtypeStruct((Np, 1), jnp.float32),
    )(degp)


def _sage_layer(h2, aggp, invd, WnT, WsT, b2d, relu, wuv=None, buv=None):
    _, N, Dh = h2.shape
    D = 2 * Dh
    H = WsT.shape[1]
    BN = 2000
    assert N % BN == 0
    G = N // BN

    def body(h_ref, agg_ref, invd_ref, wn_ref, ws_ref, b_ref, *rest):
        h = jnp.concatenate([h_ref[0], h_ref[1]], axis=-1)
        agg = jnp.concatenate([agg_ref[0], agg_ref[1]], axis=-1)
        hn = agg * invd_ref[...]
        o = (jnp.dot(h, ws_ref[...], preferred_element_type=jnp.float32)
             + jnp.dot(hn, wn_ref[...], preferred_element_type=jnp.float32)
             + b_ref[...])
        if relu:
            o = jnp.maximum(o, 0.0)
        if wuv is None:
            o_ref, = rest
            o_ref[0] = o[:, :Dh]
            o_ref[1] = o[:, Dh:]
        else:
            wuv_ref, buv_ref, uv_ref = rest
            uv_ref[...] = (jnp.dot(o, wuv_ref[...],
                                   preferred_element_type=jnp.float32)
                           + buv_ref[...])

    in_specs = [
        pl.BlockSpec((NC, BN, Dh), lambda i: (0, i, 0)),
        pl.BlockSpec((NC, BN, Dh), lambda i: (0, i, 0)),
        pl.BlockSpec((BN, 1), lambda i: (i, 0)),
        pl.BlockSpec((D, H), lambda i: (0, 0)),
        pl.BlockSpec((D, H), lambda i: (0, 0)),
        pl.BlockSpec((1, H), lambda i: (0, 0)),
    ]
    args = [h2, aggp, invd, WnT, WsT, b2d]
    if wuv is None:
        out_specs = pl.BlockSpec((NC, BN, Dh), lambda i: (0, i, 0))
        out_shape = jax.ShapeDtypeStruct((NC, N, Dh), jnp.float32)
    else:
        in_specs += [
            pl.BlockSpec((H, 2), lambda i: (0, 0)),
            pl.BlockSpec((1, 2), lambda i: (0, 0)),
        ]
        out_specs = pl.BlockSpec((BN, 2), lambda i: (i, 0))
        out_shape = jax.ShapeDtypeStruct((N, 2), jnp.float32)
        args += [wuv, buv]

    return pl.pallas_call(
        body, grid=(G,), in_specs=in_specs,
        out_specs=out_specs, out_shape=out_shape,
    )(*args)


def kernel(x, edge_index, W1_neigh, W1_self, b1, W2_neigh, W2_self, b2, Wp, bp):
    N, D = x.shape
    E = edge_index.shape[1]
    H = W1_neigh.shape[0]
    O = W2_neigh.shape[0]
    Dh = D // 2

    Np = -(-(N + 1) // (NS * 8)) * (NS * 8)
    EP = -(-E // 2048) * 2048
    pad = EP - E

    src = edge_index[0]
    dst = edge_index[1]
    src_p = jnp.concatenate([src, jnp.zeros((pad,), jnp.int32)])
    dst_p = jnp.concatenate([dst, jnp.full((pad,), N, jnp.int32)])

    x2 = jnp.stack([x[:, :Dh], x[:, Dh:]])

    seg = _make_seg_kernel(N, Np, Dh, EP)
    score_k = _make_score_kernel(N, E)

    wuv = jnp.stack([Wp[0, :O], Wp[0, O:]], axis=1)
    buv = jnp.stack([bp[0], jnp.zeros((), jnp.float32)]).reshape(1, 2)

    agg1p, degp = seg(x2, src_p, dst_p)
    invd = _deg_reduce(degp)
    h1p = _sage_layer(x2, agg1p, invd, W1_neigh.T, W1_self.T,
                      b1.reshape(1, H), relu=True)
    agg2p, _ = seg(h1p, src_p, dst_p)
    uv = _sage_layer(h1p, agg2p, invd, W2_neigh.T, W2_self.T,
                     b2.reshape(1, O), relu=False, wuv=wuv, buv=buv)
    out = score_k(uv, src, dst)
    return out.reshape(E, 1)


# speedup vs baseline: 1.0449x; 1.0449x over previous
import functools

import jax
import jax.numpy as jnp
from jax import lax
from jax.experimental import pallas as pl
from jax.experimental.pallas import tpu as pltpu
from jax.experimental.pallas import tpu_sc as plsc

NC = 2
NS = 16
L = 16
B = 128


@functools.lru_cache(maxsize=None)
def _make_seg_kernel(N, Np, Dh, EP):
    RPT = Np // NS
    RA = RPT + 8
    CH = 2048
    NCH = EP // CH
    assert EP % CH == 0
    LCAP = CH + B + 2 * L
    mesh = plsc.VectorSubcoreMesh(
        core_axis_name="c", subcore_axis_name="s",
        num_cores=NC, num_subcores=NS)

    @functools.partial(
        pl.kernel,
        out_type=[
            jax.ShapeDtypeStruct((NC, Np, Dh), jnp.float32),
            jax.ShapeDtypeStruct((NC, Np), jnp.float32),
        ],
        mesh=mesh,
        scratch_types=[
            pltpu.VMEM((2, CH), jnp.int32),
            pltpu.VMEM((2, CH), jnp.int32),
            pltpu.VMEM((LCAP,), jnp.int32),
            pltpu.VMEM((LCAP,), jnp.int32),
            pltpu.VMEM((2, B, Dh), jnp.float32),
            pltpu.VMEM((RA, Dh), jnp.float32),
            pltpu.VMEM((RA,), jnp.float32),
            pltpu.SemaphoreType.DMA,
            pltpu.SemaphoreType.DMA,
        ],
        compiler_params=pltpu.CompilerParams(
            needs_layout_passes=False, use_tc_tiling_on_sc=False),
    )
    def seg(h2_hbm, srcf_hbm, dstf_hbm, agg_hbm, deg_hbm,
            dbuf, sbuf, ldl, sgl, stage, acc, degv, sem, sem2):
        c = lax.axis_index("c")
        s = lax.axis_index("s")
        lo = s * RPT
        z16 = jnp.zeros((L,), jnp.float32)
        ones16 = jnp.ones((L,), jnp.float32)
        pad16 = jnp.full((L,), RPT + 1, jnp.int32)
        zero16i = jnp.zeros((L,), jnp.int32)

        def zero_acc(i, carry):
            for j in range(Dh // L):
                acc[i, pl.ds(j * L, L)] = z16
            return carry
        lax.fori_loop(0, RA, zero_acc, 0)

        def zero_deg(i, carry):
            degv[pl.ds(i * L, L)] = z16
            return carry
        lax.fori_loop(0, RA // L, zero_deg, 0)

        col = lax.iota(jnp.int32, L)
        cols = [col + g * L for g in range(Dh // L)]

        def issue_gather(b, buf):
            return pltpu.async_copy(
                h2_hbm.at[c].at[sgl.at[pl.ds(b * B, B)]], stage.at[buf],
                sem)

        def drain(cnt, nb):
            @pl.when(nb > 0)
            def _():
                issue_gather(0, 0)

            def batch(b, carry):
                buf = lax.rem(b, 2)
                for l in range(B // L):
                    dvec = ldl[pl.ds(b * B + l * L, L)]
                    plsc.addupdate_scatter(degv, [dvec], ones16)
                pltpu.make_async_copy(
                    h2_hbm.at[c].at[sgl.at[pl.ds(0, B)]], stage.at[buf],
                    sem).wait()

                @pl.when(b + 1 < nb)
                def _():
                    issue_gather(b + 1, 1 - buf)

                ebase = jnp.full((L,), b * B, jnp.int32)

                def edge(i, carry2):
                    for u in range(4):
                        e = i * 4 + u
                        rvec = plsc.load_gather(ldl, [ebase + e])
                        for g in range(Dh // L):
                            vals = stage[buf, e, pl.ds(g * L, L)]
                            plsc.addupdate_scatter(acc, [rvec, cols[g]], vals)
                    return carry2
                lax.fori_loop(0, B // 4, edge, 0)
                return carry
            lax.fori_loop(0, nb, batch, 0)
            for t in range(B // L):
                ldl[pl.ds(t * L, L)] = ldl[pl.ds(nb * B + t * L, L)]
                sgl[pl.ds(t * L, L)] = sgl[pl.ds(nb * B + t * L, L)]
            return cnt - nb * B

        def issue_chunk(ch, buf):
            pltpu.async_copy(dstf_hbm.at[pl.ds(ch * CH, CH)], dbuf.at[buf],
                             sem2)
            pltpu.async_copy(srcf_hbm.at[pl.ds(ch * CH, CH)], sbuf.at[buf],
                             sem2)

        issue_chunk(0, 0)

        def chunk(ch, cnt):
            buf = lax.rem(ch, 2)
            pltpu.make_async_copy(dstf_hbm.at[pl.ds(0, CH)], dbuf.at[buf],
                                  sem2).wait()
            pltpu.make_async_copy(srcf_hbm.at[pl.ds(0, CH)], sbuf.at[buf],
                                  sem2).wait()

            @pl.when(ch + 1 < NCH)
            def _():
                issue_chunk(ch + 1, 1 - buf)

            def scan(v, cnt):
                dvec = dbuf[buf, pl.ds(v * L, L)]
                svec = sbuf[buf, pl.ds(v * L, L)]
                ld = dvec - lo
                m = (ld >= 0) & (ld < RPT)
                plsc.store_compressed(ldl.at[pl.ds(cnt, L)], ld, mask=m)
                plsc.store_compressed(sgl.at[pl.ds(cnt, L)], svec, mask=m)
                return cnt + jnp.max(plsc.all_reduce_population_count(m))
            cnt = lax.fori_loop(0, CH // L, scan, cnt)
            return drain(cnt, cnt // B)
        cnt = lax.fori_loop(0, NCH, chunk, 0)

        for t in range(B // L):
            ldl[pl.ds(cnt + t * L, L)] = pad16
            sgl[pl.ds(cnt + t * L, L)] = zero16i
        drain(cnt, (cnt + B - 1) // B)

        pltpu.sync_copy(acc.at[pl.ds(0, RPT)],
                        agg_hbm.at[c, pl.ds(lo, RPT)])
        pltpu.sync_copy(degv.at[pl.ds(0, RPT)],
                        deg_hbm.at[c, pl.ds(lo, RPT)])

    return seg


@functools.lru_cache(maxsize=None)
def _make_score_kernel(N, E):
    NW = NC * NS
    EPT = E // NW
    assert EPT % 8 == 0 and EPT % L == 0
    mesh = plsc.VectorSubcoreMesh(
        core_axis_name="c", subcore_axis_name="s",
        num_cores=NC, num_subcores=NS)

    @functools.partial(
        pl.kernel,
        out_type=jax.ShapeDtypeStruct((E,), jnp.float32),
        mesh=mesh,
        scratch_types=[
            pltpu.VMEM((N, 2), jnp.float32),
            pltpu.VMEM((EPT,), jnp.int32),
            pltpu.VMEM((EPT,), jnp.int32),
            pltpu.VMEM((EPT,), jnp.float32),
        ],
        compiler_params=pltpu.CompilerParams(
            needs_layout_passes=False, use_tc_tiling_on_sc=False),
    )
    def score(uv_hbm, src_hbm, dst_hbm, out_hbm, uv, sv, dv, ov):
        c = lax.axis_index("c")
        s = lax.axis_index("s")
        wid = s * NC + c
        pltpu.sync_copy(uv_hbm, uv)
        pltpu.sync_copy(src_hbm.at[pl.ds(wid * EPT, EPT)], sv)
        pltpu.sync_copy(dst_hbm.at[pl.ds(wid * EPT, EPT)], dv)
        zc = jnp.zeros((L,), jnp.int32)
        oc = jnp.ones((L,), jnp.int32)

        def body(i, carry):
            si = sv[pl.ds(i * L, L)]
            di = dv[pl.ds(i * L, L)]
            us = plsc.load_gather(uv, [si, zc])
            vd = plsc.load_gather(uv, [di, oc])
            ov[pl.ds(i * L, L)] = us + vd
            return carry
        lax.fori_loop(0, EPT // L, body, 0)
        pltpu.sync_copy(ov, out_hbm.at[pl.ds(wid * EPT, EPT)])

    return score


def _deg_reduce(degp):
    NCc, Np = degp.shape

    def body(degp_ref, invd_ref):
        deg = (degp_ref[0] + degp_ref[1]) * 0.5
        invd_ref[...] = (1.0 / jnp.maximum(deg, 1.0))[:, None]

    return pl.pallas_call(
        body,
        in_specs=[pl.BlockSpec((NCc, Np), lambda: (0, 0))],
        out_specs=pl.BlockSpec((Np, 1), lambda: (0, 0)),
        out_shape=jax.ShapeDtypeStruct((Np, 1), jnp.float32),
    )(degp)


def _sage_layer(h2, aggp, invd, WnT, WsT, b2d, relu, wuv=None, buv=None):
    _, N, Dh = h2.shape
    D = 2 * Dh
    H = WsT.shape[1]
    BN = 2000
    assert N % BN == 0
    G = N // BN

    def body(h_ref, agg_ref, invd_ref, wn_ref, ws_ref, b_ref, *rest):
        h = jnp.concatenate([h_ref[0], h_ref[1]], axis=-1)
        agg = jnp.concatenate([agg_ref[0], agg_ref[1]], axis=-1)
        hn = agg * invd_ref[...]
        o = (jnp.dot(h, ws_ref[...], preferred_element_type=jnp.float32)
             + jnp.dot(hn, wn_ref[...], preferred_element_type=jnp.float32)
             + b_ref[...])
        if relu:
            o = jnp.maximum(o, 0.0)
        if wuv is None:
            o_ref, = rest
            o_ref[0] = o[:, :Dh]
            o_ref[1] = o[:, Dh:]
        else:
            wuv_ref, buv_ref, uv_ref = rest
            uv_ref[...] = (jnp.dot(o, wuv_ref[...],
                                   preferred_element_type=jnp.float32)
                           + buv_ref[...])

    in_specs = [
        pl.BlockSpec((NC, BN, Dh), lambda i: (0, i, 0)),
        pl.BlockSpec((NC, BN, Dh), lambda i: (0, i, 0)),
        pl.BlockSpec((BN, 1), lambda i: (i, 0)),
        pl.BlockSpec((D, H), lambda i: (0, 0)),
        pl.BlockSpec((D, H), lambda i: (0, 0)),
        pl.BlockSpec((1, H), lambda i: (0, 0)),
    ]
    args = [h2, aggp, invd, WnT, WsT, b2d]
    if wuv is None:
        out_specs = pl.BlockSpec((NC, BN, Dh), lambda i: (0, i, 0))
        out_shape = jax.ShapeDtypeStruct((NC, N, Dh), jnp.float32)
    else:
        in_specs += [
            pl.BlockSpec((H, 2), lambda i: (0, 0)),
            pl.BlockSpec((1, 2), lambda i: (0, 0)),
        ]
        out_specs = pl.BlockSpec((BN, 2), lambda i: (i, 0))
        out_shape = jax.ShapeDtypeStruct((N, 2), jnp.float32)
        args += [wuv, buv]

    return pl.pallas_call(
        body, grid=(G,), in_specs=in_specs,
        out_specs=out_specs, out_shape=out_shape,
    )(*args)


def kernel(x, edge_index, W1_neigh, W1_self, b1, W2_neigh, W2_self, b2, Wp, bp):
    N, D = x.shape
    E = edge_index.shape[1]
    H = W1_neigh.shape[0]
    O = W2_neigh.shape[0]
    Dh = D // 2

    Np = -(-(N + 1) // (NS * 8)) * (NS * 8)
    EP = -(-E // 2048) * 2048
    pad = EP - E

    src = edge_index[0]
    dst = edge_index[1]
    src_p = jnp.concatenate([src, jnp.zeros((pad,), jnp.int32)])
    dst_p = jnp.concatenate([dst, jnp.full((pad,), N, jnp.int32)])

    x2 = jnp.stack([x[:, :Dh], x[:, Dh:]])

    seg = _make_seg_kernel(N, Np, Dh, EP)
    score_k = _make_score_kernel(N, E)

    wuv = jnp.stack([Wp[0, :O], Wp[0, O:]], axis=1)
    buv = jnp.stack([bp[0], jnp.zeros((), jnp.float32)]).reshape(1, 2)

    agg1p, degp = seg(x2, src_p, dst_p)
    invd = _deg_reduce(degp)
    h1p = _sage_layer(x2, agg1p, invd, W1_neigh.T, W1_self.T,
                      b1.reshape(1, H), relu=True)
    agg2p, _ = seg(h1p, src_p, dst_p)
    uv = _sage_layer(h1p, agg2p, invd, W2_neigh.T, W2_self.T,
                     b2.reshape(1, O), relu=False, wuv=wuv, buv=buv)
    out = score_k(uv, src, dst)
    return out.reshape(E, 1)


# speedup vs baseline: 1.0603x; 1.0148x over previous
import functools

import jax
import jax.numpy as jnp
from jax import lax
from jax.experimental import pallas as pl
from jax.experimental.pallas import tpu as pltpu
from jax.experimental.pallas import tpu_sc as plsc

NC = 2
NS = 16
L = 16
B = 128


@functools.lru_cache(maxsize=None)
def _make_seg_kernel(N, Np, Dh, EP):
    RPT = Np // NS
    RA = RPT + 8
    CH = 2048
    NCH = EP // CH
    assert EP % CH == 0
    LCAP = CH + B + 2 * L
    mesh = plsc.VectorSubcoreMesh(
        core_axis_name="c", subcore_axis_name="s",
        num_cores=NC, num_subcores=NS)

    @functools.partial(
        pl.kernel,
        out_type=[
            jax.ShapeDtypeStruct((NC, Np, Dh), jnp.float32),
            jax.ShapeDtypeStruct((NC, Np), jnp.float32),
        ],
        mesh=mesh,
        scratch_types=[
            pltpu.VMEM((2, CH), jnp.int32),
            pltpu.VMEM((2, CH), jnp.int32),
            pltpu.VMEM((LCAP,), jnp.int32),
            pltpu.VMEM((LCAP,), jnp.int32),
            pltpu.VMEM((2, B, Dh), jnp.float32),
            pltpu.VMEM((RA, Dh), jnp.float32),
            pltpu.VMEM((RA,), jnp.float32),
            pltpu.SemaphoreType.DMA,
            pltpu.SemaphoreType.DMA,
        ],
        compiler_params=pltpu.CompilerParams(
            needs_layout_passes=False, use_tc_tiling_on_sc=False),
    )
    def seg(h2_hbm, srcf_hbm, dstf_hbm, agg_hbm, deg_hbm,
            dbuf, sbuf, ldl, sgl, stage, acc, degv, sem, sem2):
        c = lax.axis_index("c")
        s = lax.axis_index("s")
        lo = s * RPT
        z16 = jnp.zeros((L,), jnp.float32)
        ones16 = jnp.ones((L,), jnp.float32)
        pad16 = jnp.full((L,), RPT + 1, jnp.int32)
        zero16i = jnp.zeros((L,), jnp.int32)

        def zero_acc(i, carry):
            for j in range(Dh // L):
                acc[i, pl.ds(j * L, L)] = z16
            return carry
        lax.fori_loop(0, RA, zero_acc, 0)

        def zero_deg(i, carry):
            degv[pl.ds(i * L, L)] = z16
            return carry
        lax.fori_loop(0, RA // L, zero_deg, 0)

        col = lax.iota(jnp.int32, L)
        cols = [col + g * L for g in range(Dh // L)]

        def issue_gather(b, buf):
            return pltpu.async_copy(
                h2_hbm.at[c].at[sgl.at[pl.ds(b * B, B)]], stage.at[buf],
                sem)

        def drain(cnt, nb):
            @pl.when(nb > 0)
            def _():
                issue_gather(0, 0)

            def batch(b, carry):
                buf = lax.rem(b, 2)
                for l in range(B // L):
                    dvec = ldl[pl.ds(b * B + l * L, L)]
                    plsc.addupdate_scatter(degv, [dvec], ones16)
                pltpu.make_async_copy(
                    h2_hbm.at[c].at[sgl.at[pl.ds(0, B)]], stage.at[buf],
                    sem).wait()

                @pl.when(b + 1 < nb)
                def _():
                    issue_gather(b + 1, 1 - buf)

                ebase = jnp.full((L,), b * B, jnp.int32)

                def edge(i, carry2):
                    for u in range(8):
                        e = i * 8 + u
                        rvec = plsc.load_gather(ldl, [ebase + e])
                        for g in range(Dh // L):
                            vals = stage[buf, e, pl.ds(g * L, L)]
                            plsc.addupdate_scatter(acc, [rvec, cols[g]], vals)
                    return carry2
                lax.fori_loop(0, B // 8, edge, 0)
                return carry
            lax.fori_loop(0, nb, batch, 0)
            for t in range(B // L):
                ldl[pl.ds(t * L, L)] = ldl[pl.ds(nb * B + t * L, L)]
                sgl[pl.ds(t * L, L)] = sgl[pl.ds(nb * B + t * L, L)]
            return cnt - nb * B

        def issue_chunk(ch, buf):
            pltpu.async_copy(dstf_hbm.at[pl.ds(ch * CH, CH)], dbuf.at[buf],
                             sem2)
            pltpu.async_copy(srcf_hbm.at[pl.ds(ch * CH, CH)], sbuf.at[buf],
                             sem2)

        issue_chunk(0, 0)

        def chunk(ch, cnt):
            buf = lax.rem(ch, 2)
            pltpu.make_async_copy(dstf_hbm.at[pl.ds(0, CH)], dbuf.at[buf],
                                  sem2).wait()
            pltpu.make_async_copy(srcf_hbm.at[pl.ds(0, CH)], sbuf.at[buf],
                                  sem2).wait()

            @pl.when(ch + 1 < NCH)
            def _():
                issue_chunk(ch + 1, 1 - buf)

            def scan(v, cnt):
                dvec = dbuf[buf, pl.ds(v * L, L)]
                svec = sbuf[buf, pl.ds(v * L, L)]
                ld = dvec - lo
                m = (ld >= 0) & (ld < RPT)
                plsc.store_compressed(ldl.at[pl.ds(cnt, L)], ld, mask=m)
                plsc.store_compressed(sgl.at[pl.ds(cnt, L)], svec, mask=m)
                return cnt + jnp.max(plsc.all_reduce_population_count(m))
            cnt = lax.fori_loop(0, CH // L, scan, cnt)
            return drain(cnt, cnt // B)
        cnt = lax.fori_loop(0, NCH, chunk, 0)

        for t in range(B // L):
            ldl[pl.ds(cnt + t * L, L)] = pad16
            sgl[pl.ds(cnt + t * L, L)] = zero16i
        drain(cnt, (cnt + B - 1) // B)

        pltpu.sync_copy(acc.at[pl.ds(0, RPT)],
                        agg_hbm.at[c, pl.ds(lo, RPT)])
        pltpu.sync_copy(degv.at[pl.ds(0, RPT)],
                        deg_hbm.at[c, pl.ds(lo, RPT)])

    return seg


@functools.lru_cache(maxsize=None)
def _make_score_kernel(N, E):
    NW = NC * NS
    EPT = E // NW
    assert EPT % 8 == 0 and EPT % L == 0
    mesh = plsc.VectorSubcoreMesh(
        core_axis_name="c", subcore_axis_name="s",
        num_cores=NC, num_subcores=NS)

    @functools.partial(
        pl.kernel,
        out_type=jax.ShapeDtypeStruct((E,), jnp.float32),
        mesh=mesh,
        scratch_types=[
            pltpu.VMEM((N, 2), jnp.float32),
            pltpu.VMEM((EPT,), jnp.int32),
            pltpu.VMEM((EPT,), jnp.int32),
            pltpu.VMEM((EPT,), jnp.float32),
        ],
        compiler_params=pltpu.CompilerParams(
            needs_layout_passes=False, use_tc_tiling_on_sc=False),
    )
    def score(uv_hbm, src_hbm, dst_hbm, out_hbm, uv, sv, dv, ov):
        c = lax.axis_index("c")
        s = lax.axis_index("s")
        wid = s * NC + c
        pltpu.sync_copy(uv_hbm, uv)
        pltpu.sync_copy(src_hbm.at[pl.ds(wid * EPT, EPT)], sv)
        pltpu.sync_copy(dst_hbm.at[pl.ds(wid * EPT, EPT)], dv)
        zc = jnp.zeros((L,), jnp.int32)
        oc = jnp.ones((L,), jnp.int32)

        def body(i, carry):
            si = sv[pl.ds(i * L, L)]
            di = dv[pl.ds(i * L, L)]
            us = plsc.load_gather(uv, [si, zc])
            vd = plsc.load_gather(uv, [di, oc])
            ov[pl.ds(i * L, L)] = us + vd
            return carry
        lax.fori_loop(0, EPT // L, body, 0)
        pltpu.sync_copy(ov, out_hbm.at[pl.ds(wid * EPT, EPT)])

    return score


def _deg_reduce(degp):
    NCc, Np = degp.shape

    def body(degp_ref, invd_ref):
        deg = (degp_ref[0] + degp_ref[1]) * 0.5
        invd_ref[...] = (1.0 / jnp.maximum(deg, 1.0))[:, None]

    return pl.pallas_call(
        body,
        in_specs=[pl.BlockSpec((NCc, Np), lambda: (0, 0))],
        out_specs=pl.BlockSpec((Np, 1), lambda: (0, 0)),
        out_shape=jax.ShapeDtypeStruct((Np, 1), jnp.float32),
    )(degp)


def _sage_layer(h2, aggp, invd, WnT, WsT, b2d, relu, wuv=None, buv=None):
    _, N, Dh = h2.shape
    D = 2 * Dh
    H = WsT.shape[1]
    BN = 2000
    assert N % BN == 0
    G = N // BN

    def body(h_ref, agg_ref, invd_ref, wn_ref, ws_ref, b_ref, *rest):
        h = jnp.concatenate([h_ref[0], h_ref[1]], axis=-1)
        agg = jnp.concatenate([agg_ref[0], agg_ref[1]], axis=-1)
        hn = agg * invd_ref[...]
        o = (jnp.dot(h, ws_ref[...], preferred_element_type=jnp.float32)
             + jnp.dot(hn, wn_ref[...], preferred_element_type=jnp.float32)
             + b_ref[...])
        if relu:
            o = jnp.maximum(o, 0.0)
        if wuv is None:
            o_ref, = rest
            o_ref[0] = o[:, :Dh]
            o_ref[1] = o[:, Dh:]
        else:
            wuv_ref, buv_ref, uv_ref = rest
            uv_ref[...] = (jnp.dot(o, wuv_ref[...],
                                   preferred_element_type=jnp.float32)
                           + buv_ref[...])

    in_specs = [
        pl.BlockSpec((NC, BN, Dh), lambda i: (0, i, 0)),
        pl.BlockSpec((NC, BN, Dh), lambda i: (0, i, 0)),
        pl.BlockSpec((BN, 1), lambda i: (i, 0)),
        pl.BlockSpec((D, H), lambda i: (0, 0)),
        pl.BlockSpec((D, H), lambda i: (0, 0)),
        pl.BlockSpec((1, H), lambda i: (0, 0)),
    ]
    args = [h2, aggp, invd, WnT, WsT, b2d]
    if wuv is None:
        out_specs = pl.BlockSpec((NC, BN, Dh), lambda i: (0, i, 0))
        out_shape = jax.ShapeDtypeStruct((NC, N, Dh), jnp.float32)
    else:
        in_specs += [
            pl.BlockSpec((H, 2), lambda i: (0, 0)),
            pl.BlockSpec((1, 2), lambda i: (0, 0)),
        ]
        out_specs = pl.BlockSpec((BN, 2), lambda i: (i, 0))
        out_shape = jax.ShapeDtypeStruct((N, 2), jnp.float32)
        args += [wuv, buv]

    return pl.pallas_call(
        body, grid=(G,), in_specs=in_specs,
        out_specs=out_specs, out_shape=out_shape,
    )(*args)


def kernel(x, edge_index, W1_neigh, W1_self, b1, W2_neigh, W2_self, b2, Wp, bp):
    N, D = x.shape
    E = edge_index.shape[1]
    H = W1_neigh.shape[0]
    O = W2_neigh.shape[0]
    Dh = D // 2

    Np = -(-(N + 1) // (NS * 8)) * (NS * 8)
    EP = -(-E // 2048) * 2048
    pad = EP - E

    src = edge_index[0]
    dst = edge_index[1]
    src_p = jnp.concatenate([src, jnp.zeros((pad,), jnp.int32)])
    dst_p = jnp.concatenate([dst, jnp.full((pad,), N, jnp.int32)])

    x2 = jnp.stack([x[:, :Dh], x[:, Dh:]])

    seg = _make_seg_kernel(N, Np, Dh, EP)
    score_k = _make_score_kernel(N, E)

    wuv = jnp.stack([Wp[0, :O], Wp[0, O:]], axis=1)
    buv = jnp.stack([bp[0], jnp.zeros((), jnp.float32)]).reshape(1, 2)

    agg1p, degp = seg(x2, src_p, dst_p)
    invd = _deg_reduce(degp)
    h1p = _sage_layer(x2, agg1p, invd, W1_neigh.T, W1_self.T,
                      b1.reshape(1, H), relu=True)
    agg2p, _ = seg(h1p, src_p, dst_p)
    uv = _sage_layer(h1p, agg2p, invd, W2_neigh.T, W2_self.T,
                     b2.reshape(1, O), relu=False, wuv=wuv, buv=buv)
    out = score_k(uv, src, dst)
    return out.reshape(E, 1)


# speedup vs baseline: 1.0710x; 1.0100x over previous
import functools

import jax
import jax.numpy as jnp
from jax import lax
from jax.experimental import pallas as pl
from jax.experimental.pallas import tpu as pltpu
from jax.experimental.pallas import tpu_sc as plsc

NC = 2
NS = 16
L = 16
B = 128


@functools.lru_cache(maxsize=None)
def _make_seg_kernel(N, Np, Dh, EP):
    RPT = Np // NS
    RA = RPT + 8
    CH = 2048
    NCH = EP // CH
    assert EP % CH == 0
    LCAP = CH + B + 2 * L
    mesh = plsc.VectorSubcoreMesh(
        core_axis_name="c", subcore_axis_name="s",
        num_cores=NC, num_subcores=NS)

    @functools.partial(
        pl.kernel,
        out_type=[
            jax.ShapeDtypeStruct((NC, Np, Dh), jnp.float32),
            jax.ShapeDtypeStruct((NC, Np), jnp.float32),
        ],
        mesh=mesh,
        scratch_types=[
            pltpu.VMEM((2, CH), jnp.int32),
            pltpu.VMEM((2, CH), jnp.int32),
            pltpu.VMEM((LCAP,), jnp.int32),
            pltpu.VMEM((LCAP,), jnp.int32),
            pltpu.VMEM((2, B, Dh), jnp.float32),
            pltpu.VMEM((RA, Dh), jnp.float32),
            pltpu.VMEM((RA,), jnp.float32),
            pltpu.SemaphoreType.DMA,
            pltpu.SemaphoreType.DMA,
        ],
        compiler_params=pltpu.CompilerParams(
            needs_layout_passes=False, use_tc_tiling_on_sc=False),
    )
    def seg(h2_hbm, srcf_hbm, dstf_hbm, agg_hbm, deg_hbm,
            dbuf, sbuf, ldl, sgl, stage, acc, degv, sem, sem2):
        c = lax.axis_index("c")
        s = lax.axis_index("s")
        lo = s * RPT
        z16 = jnp.zeros((L,), jnp.float32)
        ones16 = jnp.ones((L,), jnp.float32)
        pad16 = jnp.full((L,), RPT + 1, jnp.int32)
        zero16i = jnp.zeros((L,), jnp.int32)

        def zero_acc(i, carry):
            for j in range(Dh // L):
                acc[i, pl.ds(j * L, L)] = z16
            return carry
        lax.fori_loop(0, RA, zero_acc, 0)

        def zero_deg(i, carry):
            degv[pl.ds(i * L, L)] = z16
            return carry
        lax.fori_loop(0, RA // L, zero_deg, 0)

        col = lax.iota(jnp.int32, L)
        cols = [col + g * L for g in range(Dh // L)]

        def issue_gather(b, buf):
            return pltpu.async_copy(
                h2_hbm.at[c].at[sgl.at[pl.ds(b * B, B)]], stage.at[buf],
                sem)

        def drain(cnt, nb):
            @pl.when(nb > 0)
            def _():
                issue_gather(0, 0)

            def batch(b, carry):
                buf = lax.rem(b, 2)
                for l in range(B // L):
                    dvec = ldl[pl.ds(b * B + l * L, L)]
                    plsc.addupdate_scatter(degv, [dvec], ones16)
                pltpu.make_async_copy(
                    h2_hbm.at[c].at[sgl.at[pl.ds(0, B)]], stage.at[buf],
                    sem).wait()

                @pl.when(b + 1 < nb)
                def _():
                    issue_gather(b + 1, 1 - buf)

                ebase = jnp.full((L,), b * B, jnp.int32)

                def edge(i, carry2):
                    for u in range(16):
                        e = i * 16 + u
                        rvec = plsc.load_gather(ldl, [ebase + e])
                        for g in range(Dh // L):
                            vals = stage[buf, e, pl.ds(g * L, L)]
                            plsc.addupdate_scatter(acc, [rvec, cols[g]], vals)
                    return carry2
                lax.fori_loop(0, B // 16, edge, 0)
                return carry
            lax.fori_loop(0, nb, batch, 0)
            for t in range(B // L):
                ldl[pl.ds(t * L, L)] = ldl[pl.ds(nb * B + t * L, L)]
                sgl[pl.ds(t * L, L)] = sgl[pl.ds(nb * B + t * L, L)]
            return cnt - nb * B

        def issue_chunk(ch, buf):
            pltpu.async_copy(dstf_hbm.at[pl.ds(ch * CH, CH)], dbuf.at[buf],
                             sem2)
            pltpu.async_copy(srcf_hbm.at[pl.ds(ch * CH, CH)], sbuf.at[buf],
                             sem2)

        issue_chunk(0, 0)

        def chunk(ch, cnt):
            buf = lax.rem(ch, 2)
            pltpu.make_async_copy(dstf_hbm.at[pl.ds(0, CH)], dbuf.at[buf],
                                  sem2).wait()
            pltpu.make_async_copy(srcf_hbm.at[pl.ds(0, CH)], sbuf.at[buf],
                                  sem2).wait()

            @pl.when(ch + 1 < NCH)
            def _():
                issue_chunk(ch + 1, 1 - buf)

            def scan(v, cnt):
                dvec = dbuf[buf, pl.ds(v * L, L)]
                svec = sbuf[buf, pl.ds(v * L, L)]
                ld = dvec - lo
                m = (ld >= 0) & (ld < RPT)
                plsc.store_compressed(ldl.at[pl.ds(cnt, L)], ld, mask=m)
                plsc.store_compressed(sgl.at[pl.ds(cnt, L)], svec, mask=m)
                return cnt + jnp.max(plsc.all_reduce_population_count(m))
            cnt = lax.fori_loop(0, CH // L, scan, cnt)
            return drain(cnt, cnt // B)
        cnt = lax.fori_loop(0, NCH, chunk, 0)

        for t in range(B // L):
            ldl[pl.ds(cnt + t * L, L)] = pad16
            sgl[pl.ds(cnt + t * L, L)] = zero16i
        drain(cnt, (cnt + B - 1) // B)

        pltpu.sync_copy(acc.at[pl.ds(0, RPT)],
                        agg_hbm.at[c, pl.ds(lo, RPT)])
        pltpu.sync_copy(degv.at[pl.ds(0, RPT)],
                        deg_hbm.at[c, pl.ds(lo, RPT)])

    return seg


@functools.lru_cache(maxsize=None)
def _make_score_kernel(N, E):
    NW = NC * NS
    EPT = E // NW
    assert EPT % 8 == 0 and EPT % L == 0
    mesh = plsc.VectorSubcoreMesh(
        core_axis_name="c", subcore_axis_name="s",
        num_cores=NC, num_subcores=NS)

    @functools.partial(
        pl.kernel,
        out_type=jax.ShapeDtypeStruct((E,), jnp.float32),
        mesh=mesh,
        scratch_types=[
            pltpu.VMEM((N, 2), jnp.float32),
            pltpu.VMEM((EPT,), jnp.int32),
            pltpu.VMEM((EPT,), jnp.int32),
            pltpu.VMEM((EPT,), jnp.float32),
        ],
        compiler_params=pltpu.CompilerParams(
            needs_layout_passes=False, use_tc_tiling_on_sc=False),
    )
    def score(uv_hbm, src_hbm, dst_hbm, out_hbm, uv, sv, dv, ov):
        c = lax.axis_index("c")
        s = lax.axis_index("s")
        wid = s * NC + c
        pltpu.sync_copy(uv_hbm, uv)
        pltpu.sync_copy(src_hbm.at[pl.ds(wid * EPT, EPT)], sv)
        pltpu.sync_copy(dst_hbm.at[pl.ds(wid * EPT, EPT)], dv)
        zc = jnp.zeros((L,), jnp.int32)
        oc = jnp.ones((L,), jnp.int32)

        def body(i, carry):
            si = sv[pl.ds(i * L, L)]
            di = dv[pl.ds(i * L, L)]
            us = plsc.load_gather(uv, [si, zc])
            vd = plsc.load_gather(uv, [di, oc])
            ov[pl.ds(i * L, L)] = us + vd
            return carry
        lax.fori_loop(0, EPT // L, body, 0)
        pltpu.sync_copy(ov, out_hbm.at[pl.ds(wid * EPT, EPT)])

    return score


def _deg_reduce(degp):
    NCc, Np = degp.shape

    def body(degp_ref, invd_ref):
        deg = (degp_ref[0] + degp_ref[1]) * 0.5
        invd_ref[...] = (1.0 / jnp.maximum(deg, 1.0))[:, None]

    return pl.pallas_call(
        body,
        in_specs=[pl.BlockSpec((NCc, Np), lambda: (0, 0))],
        out_specs=pl.BlockSpec((Np, 1), lambda: (0, 0)),
        out_shape=jax.ShapeDtypeStruct((Np, 1), jnp.float32),
    )(degp)


def _sage_layer(h2, aggp, invd, WnT, WsT, b2d, relu, wuv=None, buv=None):
    _, N, Dh = h2.shape
    D = 2 * Dh
    H = WsT.shape[1]
    BN = 2000
    assert N % BN == 0
    G = N // BN

    def body(h_ref, agg_ref, invd_ref, wn_ref, ws_ref, b_ref, *rest):
        h = jnp.concatenate([h_ref[0], h_ref[1]], axis=-1)
        agg = jnp.concatenate([agg_ref[0], agg_ref[1]], axis=-1)
        hn = agg * invd_ref[...]
        o = (jnp.dot(h, ws_ref[...], preferred_element_type=jnp.float32)
             + jnp.dot(hn, wn_ref[...], preferred_element_type=jnp.float32)
             + b_ref[...])
        if relu:
            o = jnp.maximum(o, 0.0)
        if wuv is None:
            o_ref, = rest
            o_ref[0] = o[:, :Dh]
            o_ref[1] = o[:, Dh:]
        else:
            wuv_ref, buv_ref, uv_ref = rest
            uv_ref[...] = (jnp.dot(o, wuv_ref[...],
                                   preferred_element_type=jnp.float32)
                           + buv_ref[...])

    in_specs = [
        pl.BlockSpec((NC, BN, Dh), lambda i: (0, i, 0)),
        pl.BlockSpec((NC, BN, Dh), lambda i: (0, i, 0)),
        pl.BlockSpec((BN, 1), lambda i: (i, 0)),
        pl.BlockSpec((D, H), lambda i: (0, 0)),
        pl.BlockSpec((D, H), lambda i: (0, 0)),
        pl.BlockSpec((1, H), lambda i: (0, 0)),
    ]
    args = [h2, aggp, invd, WnT, WsT, b2d]
    if wuv is None:
        out_specs = pl.BlockSpec((NC, BN, Dh), lambda i: (0, i, 0))
        out_shape = jax.ShapeDtypeStruct((NC, N, Dh), jnp.float32)
    else:
        in_specs += [
            pl.BlockSpec((H, 2), lambda i: (0, 0)),
            pl.BlockSpec((1, 2), lambda i: (0, 0)),
        ]
        out_specs = pl.BlockSpec((BN, 2), lambda i: (i, 0))
        out_shape = jax.ShapeDtypeStruct((N, 2), jnp.float32)
        args += [wuv, buv]

    return pl.pallas_call(
        body, grid=(G,), in_specs=in_specs,
        out_specs=out_specs, out_shape=out_shape,
    )(*args)


def kernel(x, edge_index, W1_neigh, W1_self, b1, W2_neigh, W2_self, b2, Wp, bp):
    N, D = x.shape
    E = edge_index.shape[1]
    H = W1_neigh.shape[0]
    O = W2_neigh.shape[0]
    Dh = D // 2

    Np = -(-(N + 1) // (NS * 8)) * (NS * 8)
    EP = -(-E // 2048) * 2048
    pad = EP - E

    src = edge_index[0]
    dst = edge_index[1]
    src_p = jnp.concatenate([src, jnp.zeros((pad,), jnp.int32)])
    dst_p = jnp.concatenate([dst, jnp.full((pad,), N, jnp.int32)])

    x2 = jnp.stack([x[:, :Dh], x[:, Dh:]])

    seg = _make_seg_kernel(N, Np, Dh, EP)
    score_k = _make_score_kernel(N, E)

    wuv = jnp.stack([Wp[0, :O], Wp[0, O:]], axis=1)
    buv = jnp.stack([bp[0], jnp.zeros((), jnp.float32)]).reshape(1, 2)

    agg1p, degp = seg(x2, src_p, dst_p)
    invd = _deg_reduce(degp)
    h1p = _sage_layer(x2, agg1p, invd, W1_neigh.T, W1_self.T,
                      b1.reshape(1, H), relu=True)
    agg2p, _ = seg(h1p, src_p, dst_p)
    uv = _sage_layer(h1p, agg2p, invd, W2_neigh.T, W2_self.T,
                     b2.reshape(1, O), relu=False, wuv=wuv, buv=buv)
    out = score_k(uv, src, dst)
    return out.reshape(E, 1)
